# baseline (speedup 1.0000x reference)
import functools

import jax
import jax.numpy as jnp
import numpy as np
from jax import lax
from jax.experimental import pallas as pl
from jax.experimental.pallas import tpu as pltpu
from jax.experimental.pallas import tpu_sc as plsc

_LANES = 16
_NUM_WORKERS = 32
_K = 1024

_REG_LEVELS = 5
_STRIDES = (16, 8, 4, 2, 1)

_UNROLL = 14


def _tree_pivot_indices():
    def pidx(p):
        l = p.bit_length()
        k = p - (1 << (l - 1))
        return (2 * k + 1) * (1 << (10 - l)) - 1

    lvl14 = [pidx(max(j, 1)) for j in range(16)]
    lvl5 = [pidx(16 + j) for j in range(16)]
    return np.array(lvl14, np.int32), np.array(lvl5, np.int32)


def _take16(table, idx):
    return lax.gather(
        table, idx[:, None],
        dimension_numbers=lax.GatherDimensionNumbers(
            offset_dims=(), collapsed_slice_dims=(0,), start_index_map=(0,)),
        slice_sizes=(1,),
        mode=lax.GatherScatterMode.PROMISE_IN_BOUNDS)


def _sc_quantize_body(n_per_w, n_groups,
                      x_hbm, code_hbm, mt_hbm,
                      q_hbm, part_hbm,
                      x_v, q_v, code_v, mt_v, part_v,
                      sem_x0, sem_x1, sem_c, sem_t, sem_q):
    core = lax.axis_index("c")
    subcore = lax.axis_index("s")
    wid = subcore * 2 + core
    base = wid * n_per_w
    half = n_per_w // 2

    cp_x0 = pltpu.make_async_copy(x_hbm.at[pl.ds(base, half)],
                                  x_v.at[pl.ds(0, half)], sem_x0)
    cp_x1 = pltpu.make_async_copy(x_hbm.at[pl.ds(base + half, half)],
                                  x_v.at[pl.ds(half, half)], sem_x1)
    cp_c = pltpu.make_async_copy(code_hbm, code_v, sem_c)
    cp_t = pltpu.make_async_copy(mt_hbm, mt_v, sem_t)
    cp_x0.start()
    cp_c.start()
    cp_t.start()
    cp_x1.start()
    cp_x0.wait()
    cp_c.wait()
    cp_t.wait()

    tree14 = mt_v[pl.ds(_K, _LANES)]
    tree5 = mt_v[pl.ds(_K + _LANES, _LANES)]
    one = jnp.ones((_LANES,), jnp.int32)

    def group_body(g, accs):
        base_off = pl.multiple_of(g * (_LANES * _UNROLL), _LANES * _UNROLL)
        xs = [x_v[pl.ds(base_off + j * _LANES, _LANES)]
              for j in range(_UNROLL)]
        nds = [one for _ in range(_UNROLL)]
        for _ in range(4):
            pvs = [_take16(tree14, nd) for nd in nds]
            nds = [nd + nd + (pv <= xv).astype(jnp.int32)
                   for nd, pv, xv in zip(nds, pvs, xs)]
        pvs = [_take16(tree5, nd - _LANES) for nd in nds]
        nds = [nd + nd + (pv <= xv).astype(jnp.int32)
               for nd, pv, xv in zip(nds, pvs, xs)]
        cnts = [(nd - 32) * 32 for nd in nds]
        for stride in _STRIDES:
            idxs = [cnt + (stride - 1) for cnt in cnts]
            mvals = [plsc.load_gather(mt_v, [idx]) for idx in idxs]
            cnts = [jnp.where(mval <= xv, cnt + stride, cnt)
                    for cnt, mval, xv in zip(cnts, mvals, xs)]
        qs = [plsc.load_gather(code_v, [cnt]) for cnt in cnts]
        new_accs = []
        for j, (qv, xv, acc) in enumerate(zip(qs, xs, accs)):
            q_v[pl.ds(base_off + j * _LANES, _LANES)] = qv
            diff = qv - xv
            new_accs.append(acc + diff * diff)
        return tuple(new_accs)

    n_iters = n_groups // _UNROLL
    zeros = tuple(jnp.zeros((_LANES,), jnp.float32) for _ in range(_UNROLL))
    accs = lax.fori_loop(0, n_iters // 2, group_body, zeros)
    cp_q0 = pltpu.make_async_copy(q_v.at[pl.ds(0, half)],
                                  q_hbm.at[pl.ds(base, half)], sem_q)
    cp_q0.start()
    cp_x1.wait()
    accs = lax.fori_loop(n_iters // 2, n_iters, group_body, accs)
    acc = accs[0]
    for a in accs[1:]:
        acc = acc + a
    part_v[...] = acc

    cp_q0.wait()
    pltpu.sync_copy(q_v.at[pl.ds(half, half)],
                    q_hbm.at[pl.ds(base + half, half)])
    pltpu.sync_copy(part_v, part_hbm.at[wid])


def _build_sc_call(n):
    assert n % (_NUM_WORKERS * _LANES * _UNROLL) == 0
    n_per_w = n // _NUM_WORKERS
    n_groups = n_per_w // _LANES
    mesh = plsc.VectorSubcoreMesh(core_axis_name="c", subcore_axis_name="s")
    return pl.kernel(
        functools.partial(_sc_quantize_body, n_per_w, n_groups),
        out_type=(
            jax.ShapeDtypeStruct((n,), jnp.float32),
            jax.ShapeDtypeStruct((_NUM_WORKERS, _LANES), jnp.float32),
        ),
        mesh=mesh,
        scratch_types=(
            pltpu.VMEM((n_per_w,), jnp.float32),
            pltpu.VMEM((n_per_w,), jnp.float32),
            pltpu.VMEM((_K,), jnp.float32),
            pltpu.VMEM((_K + 2 * _LANES,), jnp.float32),
            pltpu.VMEM((_LANES,), jnp.float32),
            pltpu.SemaphoreType.DMA,
            pltpu.SemaphoreType.DMA,
            pltpu.SemaphoreType.DMA,
            pltpu.SemaphoreType.DMA,
            pltpu.SemaphoreType.DMA,
        ),
        compiler_params=pltpu.CompilerParams(needs_layout_passes=False),
    )


def _mt_gather_indices():
    idx14, idx5 = _tree_pivot_indices()
    a = np.concatenate([np.arange(1023), [1023], idx14, idx5]).astype(np.int32)
    b = np.concatenate([np.arange(1, 1024), [1023], idx14 + 1,
                        idx5 + 1]).astype(np.int32)
    return a, b


def kernel(pre_quantized, weight):
    b, c, h, w = pre_quantized.shape
    n = b * c * h * w
    x = pre_quantized.reshape(n)

    code = jnp.sort(weight[:, 0], stable=False)
    ga, gb = _mt_gather_indices()
    mt = (code[ga] + code[gb]) * 0.5

    q_flat, partials = _build_sc_call(n)(x, code, mt)

    loss = (jnp.sum(partials) / n) * 1.25
    quanted_out = q_flat.reshape(b, c, h, w)
    return quanted_out, loss


# speedup vs baseline: 1.2727x; 1.2727x over previous
import functools

import jax
import jax.numpy as jnp
import numpy as np
from jax import lax
from jax.experimental import pallas as pl
from jax.experimental.pallas import tpu as pltpu
from jax.experimental.pallas import tpu_sc as plsc

_LANES = 16
_NUM_WORKERS = 32
_K = 1024

_REG_LEVELS = 5
_STRIDES = (16, 8, 4, 2, 1)

_UNROLL = 14


def _tree_pivot_indices():
    def pidx(p):
        l = p.bit_length()
        k = p - (1 << (l - 1))
        return (2 * k + 1) * (1 << (10 - l)) - 1

    lvl14 = [pidx(max(j, 1)) for j in range(16)]
    lvl5 = [pidx(16 + j) for j in range(16)]
    return np.array(lvl14, np.int32), np.array(lvl5, np.int32)


def _take16(table, idx):
    return lax.gather(
        table, idx[:, None],
        dimension_numbers=lax.GatherDimensionNumbers(
            offset_dims=(), collapsed_slice_dims=(0,), start_index_map=(0,)),
        slice_sizes=(1,),
        mode=lax.GatherScatterMode.PROMISE_IN_BOUNDS)


def _sc_quantize_body(n_per_w, n_groups,
                      x_hbm, tbl_hbm,
                      q_hbm, part_hbm,
                      x_v, q_v, tbl_v, part_v,
                      sem_x, sem_t):
    core = lax.axis_index("c")
    subcore = lax.axis_index("s")
    wid = subcore * 2 + core
    base = wid * n_per_w

    cp_x = pltpu.make_async_copy(x_hbm.at[pl.ds(base, n_per_w)], x_v, sem_x)
    cp_t = pltpu.make_async_copy(tbl_hbm, tbl_v, sem_t)
    cp_x.start()
    cp_t.start()
    cp_x.wait()
    cp_t.wait()

    tree14 = tbl_v[pl.ds(2 * _K, _LANES)]
    tree5 = tbl_v[pl.ds(2 * _K + _LANES, _LANES)]
    one = jnp.ones((_LANES,), jnp.int32)

    def group_body(g, accs):
        base_off = pl.multiple_of(g * (_LANES * _UNROLL), _LANES * _UNROLL)
        xs = [x_v[pl.ds(base_off + j * _LANES, _LANES)]
              for j in range(_UNROLL)]
        nds = [one for _ in range(_UNROLL)]
        for _ in range(4):
            pvs = [_take16(tree14, nd) for nd in nds]
            nds = [nd + nd + (pv <= xv).astype(jnp.int32)
                   for nd, pv, xv in zip(nds, pvs, xs)]
        pvs = [_take16(tree5, nd - _LANES) for nd in nds]
        nds = [nd + nd + (pv <= xv).astype(jnp.int32)
               for nd, pv, xv in zip(nds, pvs, xs)]
        cnts = [(nd - 32) * 32 for nd in nds]
        for stride in _STRIDES:
            idxs = [cnt + (stride - 1 + _K) for cnt in cnts]
            mvals = [plsc.load_gather(tbl_v, [idx]) for idx in idxs]
            cnts = [jnp.where(mval <= xv, cnt + stride, cnt)
                    for cnt, mval, xv in zip(cnts, mvals, xs)]
        qs = [plsc.load_gather(tbl_v, [cnt]) for cnt in cnts]
        new_accs = []
        for j, (qv, xv, acc) in enumerate(zip(qs, xs, accs)):
            q_v[pl.ds(base_off + j * _LANES, _LANES)] = qv
            diff = qv - xv
            new_accs.append(acc + diff * diff)
        return tuple(new_accs)

    accs = lax.fori_loop(0, n_groups // _UNROLL, group_body,
                         tuple(jnp.zeros((_LANES,), jnp.float32)
                               for _ in range(_UNROLL)))
    acc = accs[0]
    for a in accs[1:]:
        acc = acc + a
    part_v[...] = acc

    pltpu.sync_copy(q_v, q_hbm.at[pl.ds(base, n_per_w)])
    pltpu.sync_copy(part_v, part_hbm.at[wid])


def _build_sc_call(n):
    assert n % (_NUM_WORKERS * _LANES * _UNROLL) == 0
    n_per_w = n // _NUM_WORKERS
    n_groups = n_per_w // _LANES
    mesh = plsc.VectorSubcoreMesh(core_axis_name="c", subcore_axis_name="s")
    return pl.kernel(
        functools.partial(_sc_quantize_body, n_per_w, n_groups),
        out_type=(
            jax.ShapeDtypeStruct((n,), jnp.float32),
            jax.ShapeDtypeStruct((_NUM_WORKERS, _LANES), jnp.float32),
        ),
        mesh=mesh,
        scratch_types=(
            pltpu.VMEM((n_per_w,), jnp.float32),
            pltpu.VMEM((n_per_w,), jnp.float32),
            pltpu.VMEM((2 * _K + 2 * _LANES,), jnp.float32),
            pltpu.VMEM((_LANES,), jnp.float32),
            pltpu.SemaphoreType.DMA,
            pltpu.SemaphoreType.DMA,
        ),
        compiler_params=pltpu.CompilerParams(needs_layout_passes=False),
    )


def kernel(pre_quantized, weight):
    b, c, h, w = pre_quantized.shape
    n = b * c * h * w
    x = pre_quantized.reshape(n)

    code = jnp.sort(weight[:, 0], stable=False)
    mid = (code[:-1] + code[1:]) * 0.5
    idx14, idx5 = _tree_pivot_indices()
    tbl = jnp.concatenate([code, mid, mid[-1:], mid[idx14], mid[idx5]])

    q_flat, partials = _build_sc_call(n)(x, tbl)

    loss = (jnp.sum(partials) / n) * 1.25
    quanted_out = q_flat.reshape(b, c, h, w)
    return quanted_out, loss
